# baseline (speedup 1.0000x reference)
import functools

import jax
import jax.numpy as jnp
from jax import lax
from jax.experimental import pallas as pl
from jax.experimental.pallas import tpu as pltpu
from jax.experimental.pallas import tpu_sc as plsc

_NUM_BOXES = 100000
_DIM = 64
_ROW = 2 * _DIM
_BATCH = 16384

_INFO = plsc.get_sparse_core_info()
_NC = _INFO.num_cores
_NS = _INFO.num_subcores
_NW = _NC * _NS
_B_PER_W = _BATCH // _NW
_NCHUNK = 4
_CH = _B_PER_W // _NCHUNK


@functools.partial(
    pl.kernel,
    out_type=jax.ShapeDtypeStruct((_BATCH, _ROW), jnp.float32),
    mesh=plsc.VectorSubcoreMesh(core_axis_name="c", subcore_axis_name="s"),
    scratch_types=[
        pltpu.VMEM((_B_PER_W,), jnp.int32),
        pltpu.VMEM((2, _CH, _ROW), jnp.float32),
        pltpu.SemaphoreType.DMA,
        pltpu.SemaphoreType.DMA,
        pltpu.SemaphoreType.DMA,
        pltpu.SemaphoreType.DMA,
    ],
)
def _gather_rows(table_hbm, ids_hbm, out_hbm, idx_v, bufs, g0, g1, s0, s1):
    wid = lax.axis_index("s") * _NC + lax.axis_index("c")
    base = wid * _B_PER_W
    gsems = (g0, g1)
    ssems = (s0, s1)
    pltpu.sync_copy(ids_hbm.at[pl.ds(base, _B_PER_W)], idx_v)

    def start_gather(c):
        return pltpu.async_copy(
            table_hbm.at[idx_v.at[pl.ds(c * _CH, _CH)]],
            bufs.at[c % 2],
            gsems[c % 2],
        )

    def start_store(c):
        return pltpu.async_copy(
            bufs.at[c % 2],
            out_hbm.at[pl.ds(base + c * _CH, _CH)],
            ssems[c % 2],
        )

    gathers = [None] * _NCHUNK
    stores = [None] * _NCHUNK
    gathers[0] = start_gather(0)
    for c in range(_NCHUNK):
        if c + 1 < _NCHUNK:
            if c >= 1:
                stores[c - 1].wait()
            gathers[c + 1] = start_gather(c + 1)
        gathers[c].wait()
        stores[c] = start_store(c)
    stores[_NCHUNK - 2].wait()
    stores[_NCHUNK - 1].wait()


def kernel(boxes, ids):
    num_models, num_boxes, two, dim = boxes.shape
    table = boxes.reshape(num_boxes, two * dim)
    ids32 = ids.astype(jnp.int32)
    out = _gather_rows(table, ids32)
    return out.reshape(num_models, _BATCH, two, dim)
